# initial kernel scaffold (unmeasured)
import jax
import jax.numpy as jnp
from jax import lax
from jax.experimental import pallas as pl
from jax.experimental.pallas import tpu as pltpu


def kernel(
    x,
):
    def body(*refs):
        pass

    out_shape = jax.ShapeDtypeStruct(..., jnp.float32)
    return pl.pallas_call(body, out_shape=out_shape)(...)


# baseline (device time: 31777 ns/iter reference)
import jax
import jax.numpy as jnp
from jax import lax
from jax.experimental import pallas as pl
from jax.experimental.pallas import tpu as pltpu

N_DEV = 32
ROWS = 512
COLS = 512
CH = ROWS // N_DEV


def kernel(x):
    def body(x_ref, out_ref, sbuf, abuf, rbuf, send1, recv1, send2, recv2):
        me = lax.axis_index("i")

        for p in range(N_DEV):
            sbuf[p] = x_ref[0, p * CH:(p + 1) * CH, :].astype(jnp.bfloat16)

        p1 = []
        for k in range(1, N_DEV):
            dst = lax.rem(me + k, N_DEV)
            rdma = pltpu.make_async_remote_copy(
                src_ref=sbuf.at[dst],
                dst_ref=abuf.at[k],
                send_sem=send1.at[k],
                recv_sem=recv1.at[k],
                device_id=(dst,),
                device_id_type=pl.DeviceIdType.MESH,
            )
            rdma.start()
            p1.append(rdma)

        acc = x_ref[0, pl.ds(me * CH, CH), :]
        for k in range(1, N_DEV):
            p1[k - 1].wait_recv()
            acc = acc + abuf[k].astype(jnp.float32)
        rbuf[...] = acc.astype(jnp.bfloat16)

        p2 = []
        for k in range(1, N_DEV):
            dst = lax.rem(me + k, N_DEV)
            rdma = pltpu.make_async_remote_copy(
                src_ref=rbuf,
                dst_ref=out_ref.at[pl.ds(me * CH, CH), :],
                send_sem=send2.at[k],
                recv_sem=recv2.at[k],
                device_id=(dst,),
                device_id_type=pl.DeviceIdType.MESH,
            )
            rdma.start()
            p2.append(rdma)

        out_ref[pl.ds(me * CH, CH), :] = rbuf[...]

        for k in range(1, N_DEV):
            p2[k - 1].wait_recv()
        for k in range(1, N_DEV):
            p1[k - 1].wait_send()
            p2[k - 1].wait_send()

    return pl.pallas_call(
        body,
        out_shape=jax.ShapeDtypeStruct((ROWS, COLS), jnp.bfloat16),
        in_specs=[pl.BlockSpec(memory_space=pltpu.VMEM)],
        out_specs=pl.BlockSpec(memory_space=pltpu.VMEM),
        scratch_shapes=[
            pltpu.VMEM((N_DEV, CH, COLS), jnp.bfloat16),
            pltpu.VMEM((N_DEV, CH, COLS), jnp.bfloat16),
            pltpu.VMEM((CH, COLS), jnp.bfloat16),
            pltpu.SemaphoreType.DMA((N_DEV,)),
            pltpu.SemaphoreType.DMA((N_DEV,)),
            pltpu.SemaphoreType.DMA((N_DEV,)),
            pltpu.SemaphoreType.DMA((N_DEV,)),
        ],
    )(x)


# device time: 26451 ns/iter; 1.2014x vs baseline; 1.2014x over previous
import jax
import jax.numpy as jnp
from jax import lax
from jax.experimental import pallas as pl
from jax.experimental.pallas import tpu as pltpu

N_DEV = 32
ROWS = 512
COLS = 512
CH = ROWS // N_DEV


def kernel(x):
    def body(x_ref, out_ref, sbuf, abuf, rbuf, send1, recv1, send2, recv2):
        me = lax.axis_index("i")

        barrier_sem = pltpu.get_barrier_semaphore()
        for k in range(1, N_DEV):
            pl.semaphore_signal(
                barrier_sem, inc=1,
                device_id=(lax.rem(me + k, N_DEV),),
                device_id_type=pl.DeviceIdType.MESH,
            )
        pl.semaphore_wait(barrier_sem, N_DEV - 1)

        p1 = []
        for k in range(1, N_DEV):
            dst = lax.rem(me + k, N_DEV)
            sbuf[k] = x_ref[0, pl.ds(dst * CH, CH), :].astype(jnp.bfloat16)
            rdma = pltpu.make_async_remote_copy(
                src_ref=sbuf.at[k],
                dst_ref=abuf.at[k],
                send_sem=send1.at[k],
                recv_sem=recv1.at[k],
                device_id=(dst,),
                device_id_type=pl.DeviceIdType.MESH,
            )
            rdma.start()
            p1.append(rdma)

        acc = x_ref[0, pl.ds(me * CH, CH), :]
        for k in range(1, N_DEV):
            p1[k - 1].wait_recv()
            acc = acc + abuf[k].astype(jnp.float32)
        rbuf[...] = acc.astype(jnp.bfloat16)

        p2 = []
        for k in range(1, N_DEV):
            dst = lax.rem(me + k, N_DEV)
            rdma = pltpu.make_async_remote_copy(
                src_ref=rbuf,
                dst_ref=out_ref.at[pl.ds(me * CH, CH), :],
                send_sem=send2.at[k],
                recv_sem=recv2.at[k],
                device_id=(dst,),
                device_id_type=pl.DeviceIdType.MESH,
            )
            rdma.start()
            p2.append(rdma)

        out_ref[pl.ds(me * CH, CH), :] = rbuf[...]

        for k in range(1, N_DEV):
            p2[k - 1].wait_recv()
        for k in range(1, N_DEV):
            p1[k - 1].wait_send()
            p2[k - 1].wait_send()

    return pl.pallas_call(
        body,
        out_shape=jax.ShapeDtypeStruct((ROWS, COLS), jnp.bfloat16),
        in_specs=[pl.BlockSpec(memory_space=pltpu.VMEM)],
        out_specs=pl.BlockSpec(memory_space=pltpu.VMEM),
        compiler_params=pltpu.CompilerParams(collective_id=0),
        scratch_shapes=[
            pltpu.VMEM((N_DEV, CH, COLS), jnp.bfloat16),
            pltpu.VMEM((N_DEV, CH, COLS), jnp.bfloat16),
            pltpu.VMEM((CH, COLS), jnp.bfloat16),
            pltpu.SemaphoreType.DMA((N_DEV,)),
            pltpu.SemaphoreType.DMA((N_DEV,)),
            pltpu.SemaphoreType.DMA((N_DEV,)),
            pltpu.SemaphoreType.DMA((N_DEV,)),
        ],
    )(x)


# device time: 4419 ns/iter; 7.1910x vs baseline; 5.9857x over previous
import os

import jax
import jax.numpy as jnp
from jax import lax
from jax.experimental import pallas as pl
from jax.experimental.pallas import tpu as pltpu

_VARIANT = os.environ.get("KERNEL_VARIANT", "full")

N_DEV = 32
ROWS = 512
COLS = 512
CH = ROWS // N_DEV
H = 2
CW = COLS // H


def kernel(x):
    def body(x_ref, out_ref, sbuf, abuf, gbuf, rbuf, send1, recv1, send2, recv2):
        me = lax.axis_index("i")

        if _VARIANT == "local":
            out_ref[...] = x_ref[0].astype(jnp.bfloat16)
            return

        if _VARIANT == "barrier1":
            bsem = pltpu.get_barrier_semaphore()
            pl.semaphore_signal(
                bsem, inc=1,
                device_id=(lax.rem(me + 1, N_DEV),),
                device_id_type=pl.DeviceIdType.MESH,
            )
            pl.semaphore_wait(bsem, 1)
            out_ref[...] = jnp.zeros((ROWS, COLS), jnp.bfloat16)
            return

        barrier_sem = pltpu.get_barrier_semaphore()
        for k in range(1, N_DEV):
            pl.semaphore_signal(
                barrier_sem, inc=1,
                device_id=(lax.rem(me + k, N_DEV),),
                device_id_type=pl.DeviceIdType.MESH,
            )
        pl.semaphore_wait(barrier_sem, N_DEV - 1)

        p1 = []
        for k in range(1, N_DEV):
            dst = lax.rem(me + k, N_DEV)
            sbuf[k] = x_ref[0, pl.ds(dst * CH, CH), :].astype(jnp.bfloat16)
            rdma = pltpu.make_async_remote_copy(
                src_ref=sbuf.at[k],
                dst_ref=abuf.at[k],
                send_sem=send1.at[0, k],
                recv_sem=recv1.at[0, k],
                device_id=(dst,),
                device_id_type=pl.DeviceIdType.MESH,
            )
            rdma.start()
            p1.append(rdma)

        acc = x_ref[0, pl.ds(me * CH, CH), :]
        for k in range(1, N_DEV):
            p1[k - 1].wait_recv()
            acc = acc + abuf[k].astype(jnp.float32)
        rbuf[...] = acc.astype(jnp.bfloat16)

        p2 = []
        for k in range(1, N_DEV):
            dst = lax.rem(me + k, N_DEV)
            rdma = pltpu.make_async_remote_copy(
                src_ref=rbuf,
                dst_ref=gbuf.at[N_DEV - k],
                send_sem=send2.at[0, k],
                recv_sem=recv2.at[0, k],
                device_id=(dst,),
                device_id_type=pl.DeviceIdType.MESH,
            )
            rdma.start()
            p2.append(rdma)

        out_ref[pl.ds(me * CH, CH), :] = rbuf[...]

        for k in range(1, N_DEV):
            p2[k - 1].wait_recv()
        for j in range(1, N_DEV):
            src_pos = lax.rem(me + j, N_DEV)
            out_ref[pl.ds(src_pos * CH, CH), :] = gbuf[j]
        for k in range(1, N_DEV):
            p1[k - 1].wait_send()
            p2[k - 1].wait_send()

    return pl.pallas_call(
        body,
        out_shape=jax.ShapeDtypeStruct((ROWS, COLS), jnp.bfloat16),
        in_specs=[pl.BlockSpec(memory_space=pltpu.VMEM)],
        out_specs=pl.BlockSpec(memory_space=pltpu.VMEM),
        compiler_params=pltpu.CompilerParams(collective_id=0),
        scratch_shapes=[
            pltpu.VMEM((N_DEV, CH, COLS), jnp.bfloat16),
            pltpu.VMEM((N_DEV, CH, COLS), jnp.bfloat16),
            pltpu.VMEM((N_DEV, CH, COLS), jnp.bfloat16),
            pltpu.VMEM((CH, COLS), jnp.bfloat16),
            pltpu.SemaphoreType.DMA((H, N_DEV)),
            pltpu.SemaphoreType.DMA((H, N_DEV)),
            pltpu.SemaphoreType.DMA((H, N_DEV)),
            pltpu.SemaphoreType.DMA((H, N_DEV)),
        ],
    )(x)
